# Initial kernel scaffold; baseline (speedup 1.0000x reference)
import functools

import jax
import jax.numpy as jnp
from jax import lax
from jax.experimental import pallas as pl
from jax.experimental.pallas import tpu as pltpu
from jax.experimental.pallas import tpu_sc as plsc

N = 10000
E = 320000
D = 128
C = 40

NC = 2
NS = 16
NW = NC * NS

NP = 10240
ROWS_PER_TILE = NP // NS
EPT = E // NW
K = 80
CHUNKS = EPT // K


def _make_spass(d_feat):
    mesh = plsc.VectorSubcoreMesh(core_axis_name="c", subcore_axis_name="s")

    @functools.partial(
        pl.kernel,
        out_type=jax.ShapeDtypeStruct((NC, NP, d_feat), jnp.float32),
        mesh=mesh,
        scratch_types=[
            pltpu.VMEM((K,), jnp.int32),
            pltpu.VMEM((K,), jnp.int32),
            pltpu.VMEM((K, d_feat), jnp.float32),
            pltpu.VMEM_SHARED((NP, d_feat), jnp.float32),
            pltpu.SemaphoreType.DMA,
        ],
    )
    def spass(h_hbm, src_hbm, dst_hbm, out_hbm, srcb, dstb, rows, acc, sem):
        c = lax.axis_index("c")
        s = lax.axis_index("s")
        wid = c * NS + s

        zv = jnp.zeros((16,), jnp.float32)

        def zrow(i, _):
            for j in range(d_feat // 16):
                rows[i, pl.ds(j * 16, 16)] = zv
            return 0

        lax.fori_loop(0, K, zrow, 0, unroll=False)

        def zacc(t, _):
            pltpu.sync_copy(rows, acc.at[pl.ds(s * ROWS_PER_TILE + t * K, K)])
            return 0

        lax.fori_loop(0, ROWS_PER_TILE // K, zacc, 0, unroll=False)
        plsc.subcore_barrier()

        base = wid * EPT

        def chunk(k, _):
            off = base + k * K
            pltpu.sync_copy(src_hbm.at[pl.ds(off, K)], srcb)
            pltpu.sync_copy(dst_hbm.at[pl.ds(off, K)], dstb)
            pltpu.async_copy(h_hbm.at[srcb], rows, sem).wait()
            pltpu.sync_copy(rows, acc.at[dstb], add=True)
            return 0

        lax.fori_loop(0, CHUNKS, chunk, 0, unroll=False)
        plsc.subcore_barrier()

        pltpu.sync_copy(
            acc.at[pl.ds(s * ROWS_PER_TILE, ROWS_PER_TILE)],
            out_hbm.at[c, pl.ds(s * ROWS_PER_TILE, ROWS_PER_TILE)],
        )

    return spass


_spass_aug = _make_spass(D + 16)
_spass = _make_spass(D)


BR = 512


def _tc_call(body, row_widths, weight_shapes, out_widths):
    grid = (NP // BR,)

    def rows_spec(w):
        return pl.BlockSpec((BR, w), lambda i: (i, 0))

    def whole_spec(shape):
        return pl.BlockSpec(shape, lambda i: tuple(0 for _ in shape))

    in_specs = [rows_spec(w) for w in row_widths] + [
        whole_spec(sh) for sh in weight_shapes
    ]
    out_specs = [rows_spec(w) for w in out_widths]
    out_shape = [jax.ShapeDtypeStruct((NP, w), jnp.float32) for w in out_widths]
    if len(out_widths) == 1:
        out_specs = out_specs[0]
        out_shape = out_shape[0]
    return pl.pallas_call(
        body, grid=grid, in_specs=in_specs, out_specs=out_specs,
        out_shape=out_shape,
    )


def _dinv(deg_ref):
    return 1.0 / jnp.maximum(deg_ref[...], 1.0)


def _tc1_body(x_ref, s1a, s1b, deg, w0, w1, b, hp1_out, part_out):
    di = _dinv(deg)
    hp1 = (s1a[...] + s1b[...]) * di
    part = (
        jnp.dot(x_ref[...], w0[...], preferred_element_type=jnp.float32)
        + jnp.dot(hp1, w1[...], preferred_element_type=jnp.float32)
        + b[...]
    )
    hp1_out[...] = hp1
    part_out[...] = part


def _layer_norm(h, g, b):
    m = jnp.mean(h, axis=-1, keepdims=True)
    v = jnp.mean((h - m) ** 2, axis=-1, keepdims=True)
    return (h - m) * lax.rsqrt(v + 1e-5) * g + b


def _tc2_body(part, s2a, s2b, deg, w2, g, b, h_out):
    di = _dinv(deg)
    hp2 = (s2a[...] + s2b[...]) * di
    h = part[...] + jnp.dot(hp2, w2[...], preferred_element_type=jnp.float32)
    h_out[...] = _layer_norm(h, g[...], b[...])


def _tc4_body(part, s4a, s4b, deg, w2, g, b, wlin, blin, out_ref):
    di = _dinv(deg)
    hp2 = (s4a[...] + s4b[...]) * di
    h = part[...] + jnp.dot(hp2, w2[...], preferred_element_type=jnp.float32)
    h = _layer_norm(h, g[...], b[...])
    logits = jnp.dot(h, wlin[...], preferred_element_type=jnp.float32) + blin[...]
    m = jnp.max(logits, axis=-1, keepdims=True)
    sh = logits - m
    lse = jnp.log(jnp.sum(jnp.exp(sh), axis=-1, keepdims=True))
    out_ref[...] = (sh - lse)[:, :C]


def kernel(x, edge_index, W0, b0, W1, b1, ln0_g, ln0_b, ln1_g, ln1_b, Wlin, blin):
    src = edge_index[0]
    dst = edge_index[1]

    x_aug = jnp.concatenate([x, jnp.ones((N, 16), jnp.float32)], axis=1)
    partsA = _spass_aug(x_aug, src, dst)
    s1a, s1b = partsA[0, :, :D], partsA[1, :, :D]
    deg = (partsA[0, :, D] + partsA[1, :, D]).reshape(NP, 1)

    x_p = jnp.pad(x, ((0, NP - N), (0, 0)))
    b0r = b0.reshape(1, D)
    b1r = b1.reshape(1, D)

    tc1 = _tc_call(_tc1_body, [D, D, D, 1],
                   [(D, D), (D, D), (1, D)], [D, D])
    hp1, part0 = tc1(x_p, s1a, s1b, deg, W0[0], W0[1], b0r)

    partsB = _spass(hp1, src, dst)
    tc2 = _tc_call(_tc2_body, [D, D, D, 1],
                   [(D, D), (1, D), (1, D)], [D])
    h0 = tc2(part0, partsB[0], partsB[1], deg, W0[2],
             ln0_g.reshape(1, D), ln0_b.reshape(1, D))

    partsC = _spass(h0, src, dst)
    tc3 = _tc_call(_tc1_body, [D, D, D, 1],
                   [(D, D), (D, D), (1, D)], [D, D])
    hp1b, part1 = tc3(h0, partsC[0], partsC[1], deg, W1[0], W1[1], b1r)

    partsD = _spass(hp1b, src, dst)
    wlin_pad = jnp.pad(Wlin, ((0, 0), (0, D - C)))
    blin_pad = jnp.concatenate(
        [blin, jnp.full((D - C,), -1e30, jnp.float32)]).reshape(1, D)
    tc4 = _tc_call(_tc4_body, [D, D, D, 1],
                   [(D, D), (1, D), (1, D), (D, D), (1, D)], [C])
    out = tc4(part1, partsD[0], partsD[1], deg, W1[2],
              ln1_g.reshape(1, D), ln1_b.reshape(1, D), wlin_pad, blin_pad)

    return out[:N]


# speedup vs baseline: 4.2100x; 4.2100x over previous
import functools

import jax
import jax.numpy as jnp
from jax import lax
from jax.experimental import pallas as pl
from jax.experimental.pallas import tpu as pltpu
from jax.experimental.pallas import tpu_sc as plsc

N = 10000
E = 320000
D = 128
C = 40

NC = 2
NS = 16
NW = NC * NS

NP = 10240
ROWS_PER_TILE = NP // NS
EPT = E // NW
K = 80
CHUNKS = EPT // K


def _make_spass(d_feat):
    mesh = plsc.VectorSubcoreMesh(core_axis_name="c", subcore_axis_name="s")

    @functools.partial(
        pl.kernel,
        out_type=jax.ShapeDtypeStruct((NC, NP, d_feat), jnp.float32),
        mesh=mesh,
        scratch_types=[
            pltpu.VMEM((K,), jnp.int32),
            pltpu.VMEM((K,), jnp.int32),
            pltpu.VMEM((K, d_feat), jnp.float32),
            pltpu.VMEM_SHARED((NP, d_feat), jnp.float32),
            pltpu.SemaphoreType.DMA,
        ],
    )
    def spass(h_hbm, src_hbm, dst_hbm, out_hbm, srcb, dstb, rows, acc, sem):
        c = lax.axis_index("c")
        s = lax.axis_index("s")
        wid = c * NS + s

        zv = jnp.zeros((16,), jnp.float32)

        def zrow(i, _):
            for j in range(d_feat // 16):
                rows[i, pl.ds(j * 16, 16)] = zv
            return 0

        lax.fori_loop(0, K, zrow, 0, unroll=False)

        def zacc(t, _):
            pltpu.sync_copy(rows, acc.at[pl.ds(s * ROWS_PER_TILE + t * K, K)])
            return 0

        lax.fori_loop(0, ROWS_PER_TILE // K, zacc, 0, unroll=False)
        plsc.subcore_barrier()

        base = wid * EPT

        def chunk(k, _):
            off = base + k * K
            pltpu.sync_copy(src_hbm.at[pl.ds(off, K)], srcb)
            pltpu.sync_copy(dst_hbm.at[pl.ds(off, K)], dstb)
            pltpu.async_copy(h_hbm.at[srcb], rows, sem).wait()
            pltpu.sync_copy(rows, acc.at[dstb], add=True)
            return 0

        lax.fori_loop(0, CHUNKS, chunk, 0, unroll=False)
        plsc.subcore_barrier()

        pltpu.sync_copy(
            acc.at[pl.ds(s * ROWS_PER_TILE, ROWS_PER_TILE)],
            out_hbm.at[c, pl.ds(s * ROWS_PER_TILE, ROWS_PER_TILE)],
        )

    return spass


_spass = _make_spass(D)

_deg_mesh = plsc.VectorSubcoreMesh(core_axis_name="c", subcore_axis_name="s")


@functools.partial(
    pl.kernel,
    out_type=jax.ShapeDtypeStruct((NC, NP, D), jnp.float32),
    mesh=_deg_mesh,
    scratch_types=[
        pltpu.VMEM((K,), jnp.int32),
        pltpu.VMEM((K, D), jnp.float32),
        pltpu.VMEM_SHARED((NP, D), jnp.float32),
    ],
)
def _spass_deg(dst_hbm, out_hbm, dstb, rows, acc):
    c = lax.axis_index("c")
    s = lax.axis_index("s")
    wid = c * NS + s

    zv = jnp.zeros((16,), jnp.float32)

    def zrow(i, _):
        for j in range(D // 16):
            rows[i, pl.ds(j * 16, 16)] = zv
        return 0

    lax.fori_loop(0, K, zrow, 0, unroll=False)

    def zacc(t, _):
        pltpu.sync_copy(rows, acc.at[pl.ds(s * ROWS_PER_TILE + t * K, K)])
        return 0

    lax.fori_loop(0, ROWS_PER_TILE // K, zacc, 0, unroll=False)

    ov = jnp.ones((16,), jnp.float32)

    def orow(i, _):
        for j in range(D // 16):
            rows[i, pl.ds(j * 16, 16)] = ov
        return 0

    lax.fori_loop(0, K, orow, 0, unroll=False)
    plsc.subcore_barrier()

    base = wid * EPT

    def chunk(k, _):
        pltpu.sync_copy(dst_hbm.at[pl.ds(base + k * K, K)], dstb)
        pltpu.sync_copy(rows, acc.at[dstb], add=True)
        return 0

    lax.fori_loop(0, CHUNKS, chunk, 0, unroll=False)
    plsc.subcore_barrier()

    pltpu.sync_copy(
        acc.at[pl.ds(s * ROWS_PER_TILE, ROWS_PER_TILE)],
        out_hbm.at[c, pl.ds(s * ROWS_PER_TILE, ROWS_PER_TILE)],
    )


BR = 512


def _tc_call(body, row_widths, weight_shapes, out_widths):
    grid = (NP // BR,)

    def rows_spec(w):
        return pl.BlockSpec((BR, w), lambda i: (i, 0))

    def whole_spec(shape):
        return pl.BlockSpec(shape, lambda i: tuple(0 for _ in shape))

    in_specs = [rows_spec(w) for w in row_widths] + [
        whole_spec(sh) for sh in weight_shapes
    ]
    out_specs = [rows_spec(w) for w in out_widths]
    out_shape = [jax.ShapeDtypeStruct((NP, w), jnp.float32) for w in out_widths]
    if len(out_widths) == 1:
        out_specs = out_specs[0]
        out_shape = out_shape[0]
    return pl.pallas_call(
        body, grid=grid, in_specs=in_specs, out_specs=out_specs,
        out_shape=out_shape,
    )


def _dinv(dega, degb):
    return 1.0 / jnp.maximum(dega[...] + degb[...], 1.0)


def _tc1_body(x_ref, s1a, s1b, dega, degb, w0, w1, b, hp1_out, part_out):
    di = _dinv(dega, degb)
    hp1 = (s1a[...] + s1b[...]) * di
    part = (
        jnp.dot(x_ref[...], w0[...], preferred_element_type=jnp.float32)
        + jnp.dot(hp1, w1[...], preferred_element_type=jnp.float32)
        + b[...]
    )
    hp1_out[...] = hp1
    part_out[...] = part


def _layer_norm(h, g, b):
    m = jnp.mean(h, axis=-1, keepdims=True)
    v = jnp.mean((h - m) ** 2, axis=-1, keepdims=True)
    return (h - m) * lax.rsqrt(v + 1e-5) * g + b


def _tc2_body(part, s2a, s2b, dega, degb, w2, g, b, h_out):
    di = _dinv(dega, degb)
    hp2 = (s2a[...] + s2b[...]) * di
    h = part[...] + jnp.dot(hp2, w2[...], preferred_element_type=jnp.float32)
    h_out[...] = _layer_norm(h, g[...], b[...])


def _tc4_body(part, s4a, s4b, dega, degb, w2, g, b, wlin, blin, out_ref):
    di = _dinv(dega, degb)
    hp2 = (s4a[...] + s4b[...]) * di
    h = part[...] + jnp.dot(hp2, w2[...], preferred_element_type=jnp.float32)
    h = _layer_norm(h, g[...], b[...])
    logits = jnp.dot(h, wlin[...], preferred_element_type=jnp.float32) + blin[...]
    m = jnp.max(logits, axis=-1, keepdims=True)
    sh = logits - m
    lse = jnp.log(jnp.sum(jnp.exp(sh), axis=-1, keepdims=True))
    out_ref[...] = (sh - lse)[:, :C]


def kernel(x, edge_index, W0, b0, W1, b1, ln0_g, ln0_b, ln1_g, ln1_b, Wlin, blin):
    src = edge_index[0]
    dst = edge_index[1]

    deg_parts = _spass_deg(dst)
    dega = deg_parts[0, :, 0:1]
    degb = deg_parts[1, :, 0:1]

    partsA = _spass(x, src, dst)
    s1a, s1b = partsA[0], partsA[1]

    x_p = jnp.pad(x, ((0, NP - N), (0, 0)))
    b0r = b0.reshape(1, D)
    b1r = b1.reshape(1, D)

    tc1 = _tc_call(_tc1_body, [D, D, D, 1, 1],
                   [(D, D), (D, D), (1, D)], [D, D])
    hp1, part0 = tc1(x_p, s1a, s1b, dega, degb, W0[0], W0[1], b0r)

    partsB = _spass(hp1, src, dst)
    tc2 = _tc_call(_tc2_body, [D, D, D, 1, 1],
                   [(D, D), (1, D), (1, D)], [D])
    h0 = tc2(part0, partsB[0], partsB[1], dega, degb, W0[2],
             ln0_g.reshape(1, D), ln0_b.reshape(1, D))

    partsC = _spass(h0, src, dst)
    tc3 = _tc_call(_tc1_body, [D, D, D, 1, 1],
                   [(D, D), (D, D), (1, D)], [D, D])
    hp1b, part1 = tc3(h0, partsC[0], partsC[1], dega, degb, W1[0], W1[1], b1r)

    partsD = _spass(hp1b, src, dst)
    wlin_pad = jnp.pad(Wlin, ((0, 0), (0, D - C)))
    blin_pad = jnp.concatenate(
        [blin, jnp.full((D - C,), -1e30, jnp.float32)]).reshape(1, D)
    tc4 = _tc_call(_tc4_body, [D, D, D, 1, 1],
                   [(D, D), (1, D), (1, D), (D, D), (1, D)], [C])
    out = tc4(part1, partsD[0], partsD[1], dega, degb, W1[2],
              ln1_g.reshape(1, D), ln1_b.reshape(1, D), wlin_pad, blin_pad)

    return out[:N]
